# baseline (speedup 1.0000x reference)
import functools

import jax
import jax.numpy as jnp
from jax.experimental import pallas as pl
from jax.experimental.pallas import tpu as pltpu


def _support_kernel(x_ref, w_ref, out_ref):
    out_ref[...] = jnp.dot(
        x_ref[...], w_ref[...], preferred_element_type=jnp.float32
    )


def _main_kernel(
    adj_ref, sup_ref, hx_ref, cx_ref, wx_ref, wh_ref, gb_ref, bias_ref,
    hy_ref, cy_ref, *, h: int
):
    acc = jnp.dot(
        adj_ref[...].astype(jnp.bfloat16), sup_ref[...],
        preferred_element_type=jnp.float32,
    )
    xs = jnp.maximum(acc, 0.0) + bias_ref[...]
    gates = (
        jnp.dot(xs, wx_ref[...], preferred_element_type=jnp.float32)
        + jnp.dot(hx_ref[...], wh_ref[...], preferred_element_type=jnp.float32)
        + gb_ref[...]
    )
    ingate = jax.nn.sigmoid(gates[:, 0:h])
    forgetgate = jax.nn.sigmoid(gates[:, h:2 * h])
    cellgate = jnp.tanh(gates[:, 2 * h:3 * h])
    outgate = jax.nn.sigmoid(gates[:, 3 * h:4 * h])
    cy = cx_ref[...] * forgetgate + ingate * cellgate
    cy_ref[...] = cy
    hy_ref[...] = outgate * jnp.tanh(cy)


@jax.jit
def kernel(x, hx, cx, adj, gcn_weight, W_x2h, b_x2h, W_h2h, b_h2h, bias):
    n, d = x.shape
    h = hx.shape[1]

    bm_s = 2000
    support = pl.pallas_call(
        _support_kernel,
        grid=(n // bm_s,),
        in_specs=[
            pl.BlockSpec((bm_s, d), lambda i: (i, 0)),
            pl.BlockSpec((d, h), lambda i: (0, 0)),
        ],
        out_specs=pl.BlockSpec((bm_s, h), lambda i: (i, 0)),
        out_shape=jax.ShapeDtypeStruct((n, h), jnp.float32),
    )(x, gcn_weight)

    wx_t = W_x2h.T
    wh_t = W_h2h.T
    gate_b = (b_x2h + b_h2h).reshape(1, 4 * h)
    bias2d = bias.reshape(1, h)

    bm = 400
    nm = n // bm

    hy, cy = pl.pallas_call(
        functools.partial(_main_kernel, h=h),
        grid=(nm,),
        in_specs=[
            pl.BlockSpec((bm, n), lambda i: (i, 0)),
            pl.BlockSpec((n, h), lambda i: (0, 0)),
            pl.BlockSpec((bm, h), lambda i: (i, 0)),
            pl.BlockSpec((bm, h), lambda i: (i, 0)),
            pl.BlockSpec((h, 4 * h), lambda i: (0, 0)),
            pl.BlockSpec((h, 4 * h), lambda i: (0, 0)),
            pl.BlockSpec((1, 4 * h), lambda i: (0, 0)),
            pl.BlockSpec((1, h), lambda i: (0, 0)),
        ],
        out_specs=[
            pl.BlockSpec((bm, h), lambda i: (i, 0)),
            pl.BlockSpec((bm, h), lambda i: (i, 0)),
        ],
        out_shape=[
            jax.ShapeDtypeStruct((n, h), jnp.float32),
            jax.ShapeDtypeStruct((n, h), jnp.float32),
        ],
        compiler_params=pltpu.CompilerParams(
            dimension_semantics=("arbitrary",),
        ),
    )(adj, support.astype(jnp.bfloat16), hx, cx, wx_t, wh_t, gate_b, bias2d)

    return (hy, cy)


# speedup vs baseline: 1.0123x; 1.0123x over previous
import functools

import jax
import jax.numpy as jnp
from jax.experimental import pallas as pl
from jax.experimental.pallas import tpu as pltpu


def _support_kernel(x_ref, w_ref, out_ref):
    out_ref[...] = jnp.dot(
        x_ref[...], w_ref[...], preferred_element_type=jnp.float32
    )


def _main_kernel(
    adj_ref, sup_ref, hx_ref, cx_ref, wx_ref, wh_ref, gb_ref, bias_ref,
    hy_ref, cy_ref, *, h: int
):
    acc = jnp.dot(
        adj_ref[...], sup_ref[...], preferred_element_type=jnp.float32
    )
    xs = jnp.maximum(acc, 0.0) + bias_ref[...]
    gates = (
        jnp.dot(xs, wx_ref[...], preferred_element_type=jnp.float32)
        + jnp.dot(hx_ref[...], wh_ref[...], preferred_element_type=jnp.float32)
        + gb_ref[...]
    )
    ingate = jax.nn.sigmoid(gates[:, 0:h])
    forgetgate = jax.nn.sigmoid(gates[:, h:2 * h])
    cellgate = jnp.tanh(gates[:, 2 * h:3 * h])
    outgate = jax.nn.sigmoid(gates[:, 3 * h:4 * h])
    cy = cx_ref[...] * forgetgate + ingate * cellgate
    cy_ref[...] = cy
    hy_ref[...] = outgate * jnp.tanh(cy)


@jax.jit
def kernel(x, hx, cx, adj, gcn_weight, W_x2h, b_x2h, W_h2h, b_h2h, bias):
    n, d = x.shape
    h = hx.shape[1]

    bm_s = 2000
    support = pl.pallas_call(
        _support_kernel,
        grid=(n // bm_s,),
        in_specs=[
            pl.BlockSpec((bm_s, d), lambda i: (i, 0)),
            pl.BlockSpec((d, h), lambda i: (0, 0)),
        ],
        out_specs=pl.BlockSpec((bm_s, h), lambda i: (i, 0)),
        out_shape=jax.ShapeDtypeStruct((n, h), jnp.float32),
    )(x, gcn_weight)

    wx_t = W_x2h.T
    wh_t = W_h2h.T
    gate_b = (b_x2h + b_h2h).reshape(1, 4 * h)
    bias2d = bias.reshape(1, h)

    bm = 200
    nm = n // bm

    hy, cy = pl.pallas_call(
        functools.partial(_main_kernel, h=h),
        grid=(nm,),
        in_specs=[
            pl.BlockSpec((bm, n), lambda i: (i, 0)),
            pl.BlockSpec((n, h), lambda i: (0, 0)),
            pl.BlockSpec((bm, h), lambda i: (i, 0)),
            pl.BlockSpec((bm, h), lambda i: (i, 0)),
            pl.BlockSpec((h, 4 * h), lambda i: (0, 0)),
            pl.BlockSpec((h, 4 * h), lambda i: (0, 0)),
            pl.BlockSpec((1, 4 * h), lambda i: (0, 0)),
            pl.BlockSpec((1, h), lambda i: (0, 0)),
        ],
        out_specs=[
            pl.BlockSpec((bm, h), lambda i: (i, 0)),
            pl.BlockSpec((bm, h), lambda i: (i, 0)),
        ],
        out_shape=[
            jax.ShapeDtypeStruct((n, h), jnp.float32),
            jax.ShapeDtypeStruct((n, h), jnp.float32),
        ],
        compiler_params=pltpu.CompilerParams(
            dimension_semantics=("arbitrary",),
        ),
    )(adj, support, hx, cx, wx_t, wh_t, gate_b, bias2d)

    return (hy, cy)


# speedup vs baseline: 1.0195x; 1.0072x over previous
import functools

import jax
import jax.numpy as jnp
from jax.experimental import pallas as pl
from jax.experimental.pallas import tpu as pltpu


def _support_kernel(x_ref, w_ref, out_ref):
    out_ref[...] = jnp.dot(
        x_ref[...], w_ref[...], preferred_element_type=jnp.float32
    )


def _main_kernel(
    adj_ref, sup_ref, hx_ref, cx_ref, wx_ref, wh_ref, gb_ref, bias_ref,
    hy_ref, cy_ref, *, h: int
):
    acc = jnp.dot(
        adj_ref[...], sup_ref[...], preferred_element_type=jnp.float32
    )
    xs = jnp.maximum(acc, 0.0) + bias_ref[...]
    gates = (
        jnp.dot(xs, wx_ref[...], preferred_element_type=jnp.float32)
        + jnp.dot(hx_ref[...], wh_ref[...], preferred_element_type=jnp.float32)
        + gb_ref[...]
    )
    ingate = jax.nn.sigmoid(gates[:, 0:h])
    forgetgate = jax.nn.sigmoid(gates[:, h:2 * h])
    cellgate = jnp.tanh(gates[:, 2 * h:3 * h])
    outgate = jax.nn.sigmoid(gates[:, 3 * h:4 * h])
    cy = cx_ref[...] * forgetgate + ingate * cellgate
    cy_ref[...] = cy
    hy_ref[...] = outgate * jnp.tanh(cy)


@jax.jit
def kernel(x, hx, cx, adj, gcn_weight, W_x2h, b_x2h, W_h2h, b_h2h, bias):
    n, d = x.shape
    h = hx.shape[1]

    bm_s = 2000
    support = pl.pallas_call(
        _support_kernel,
        grid=(n // bm_s,),
        in_specs=[
            pl.BlockSpec((bm_s, d), lambda i: (i, 0)),
            pl.BlockSpec((d, h), lambda i: (0, 0)),
        ],
        out_specs=pl.BlockSpec((bm_s, h), lambda i: (i, 0)),
        out_shape=jax.ShapeDtypeStruct((n, h), jnp.float32),
    )(x, gcn_weight)

    wx_t = W_x2h.T
    wh_t = W_h2h.T
    gate_b = (b_x2h + b_h2h).reshape(1, 4 * h)
    bias2d = bias.reshape(1, h)

    bm = 400
    nm = n // bm

    hy, cy = pl.pallas_call(
        functools.partial(_main_kernel, h=h),
        grid=(nm,),
        in_specs=[
            pl.BlockSpec((bm, n), lambda i: (i, 0)),
            pl.BlockSpec((n, h), lambda i: (0, 0)),
            pl.BlockSpec((bm, h), lambda i: (i, 0)),
            pl.BlockSpec((bm, h), lambda i: (i, 0)),
            pl.BlockSpec((h, 4 * h), lambda i: (0, 0)),
            pl.BlockSpec((h, 4 * h), lambda i: (0, 0)),
            pl.BlockSpec((1, 4 * h), lambda i: (0, 0)),
            pl.BlockSpec((1, h), lambda i: (0, 0)),
        ],
        out_specs=[
            pl.BlockSpec((bm, h), lambda i: (i, 0)),
            pl.BlockSpec((bm, h), lambda i: (i, 0)),
        ],
        out_shape=[
            jax.ShapeDtypeStruct((n, h), jnp.float32),
            jax.ShapeDtypeStruct((n, h), jnp.float32),
        ],
        compiler_params=pltpu.CompilerParams(
            dimension_semantics=("arbitrary",),
        ),
    )(adj, support, hx, cx, wx_t, wh_t, gate_b, bias2d)

    return (hy, cy)


# speedup vs baseline: 1.1021x; 1.0810x over previous
import functools

import jax
import jax.numpy as jnp
from jax.experimental import pallas as pl
from jax.experimental.pallas import tpu as pltpu


def _main_kernel(
    adj_ref, x_ref, g_ref, hx_ref, cx_ref, wx_ref, wh_ref, gb_ref, bias_ref,
    hy_ref, cy_ref, sup_ref, *, h: int
):
    @pl.when(pl.program_id(0) == 0)
    def _support():
        sup_ref[...] = jnp.dot(
            x_ref[...], g_ref[...], preferred_element_type=jnp.float32
        )

    acc = jnp.dot(
        adj_ref[...], sup_ref[...], preferred_element_type=jnp.float32
    )
    xs = jnp.maximum(acc, 0.0) + bias_ref[...]
    gates = (
        jnp.dot(xs, wx_ref[...], preferred_element_type=jnp.float32)
        + jnp.dot(hx_ref[...], wh_ref[...], preferred_element_type=jnp.float32)
        + gb_ref[...]
    )
    ingate = jax.nn.sigmoid(gates[:, 0:h])
    forgetgate = jax.nn.sigmoid(gates[:, h:2 * h])
    cellgate = jnp.tanh(gates[:, 2 * h:3 * h])
    outgate = jax.nn.sigmoid(gates[:, 3 * h:4 * h])
    cy = cx_ref[...] * forgetgate + ingate * cellgate
    cy_ref[...] = cy
    hy_ref[...] = outgate * jnp.tanh(cy)


@jax.jit
def kernel(x, hx, cx, adj, gcn_weight, W_x2h, b_x2h, W_h2h, b_h2h, bias):
    n, d = x.shape
    h = hx.shape[1]

    wx_t = W_x2h.T
    wh_t = W_h2h.T
    gate_b = (b_x2h + b_h2h).reshape(1, 4 * h)
    bias2d = bias.reshape(1, h)

    bm = 400
    nm = n // bm

    hy, cy = pl.pallas_call(
        functools.partial(_main_kernel, h=h),
        grid=(nm,),
        in_specs=[
            pl.BlockSpec((bm, n), lambda i: (i, 0)),
            pl.BlockSpec((n, d), lambda i: (0, 0)),
            pl.BlockSpec((d, h), lambda i: (0, 0)),
            pl.BlockSpec((bm, h), lambda i: (i, 0)),
            pl.BlockSpec((bm, h), lambda i: (i, 0)),
            pl.BlockSpec((h, 4 * h), lambda i: (0, 0)),
            pl.BlockSpec((h, 4 * h), lambda i: (0, 0)),
            pl.BlockSpec((1, 4 * h), lambda i: (0, 0)),
            pl.BlockSpec((1, h), lambda i: (0, 0)),
        ],
        out_specs=[
            pl.BlockSpec((bm, h), lambda i: (i, 0)),
            pl.BlockSpec((bm, h), lambda i: (i, 0)),
        ],
        out_shape=[
            jax.ShapeDtypeStruct((n, h), jnp.float32),
            jax.ShapeDtypeStruct((n, h), jnp.float32),
        ],
        scratch_shapes=[pltpu.VMEM((n, h), jnp.float32)],
        compiler_params=pltpu.CompilerParams(
            dimension_semantics=("arbitrary",),
        ),
    )(adj, x, gcn_weight, hx, cx, wx_t, wh_t, gate_b, bias2d)

    return (hy, cy)


# speedup vs baseline: 1.1042x; 1.0019x over previous
import functools

import jax
import jax.numpy as jnp
from jax.experimental import pallas as pl
from jax.experimental.pallas import tpu as pltpu


def _main_kernel(
    adj_ref, x_ref, g_ref, hx_ref, cx_ref, wx_ref, wh_ref, gb_ref, bias_ref,
    hy_ref, cy_ref, sup_ref, *, h: int
):
    @pl.when(pl.program_id(0) == 0)
    def _support():
        sup_ref[...] = jnp.dot(
            x_ref[...], g_ref[...], preferred_element_type=jnp.float32
        )

    acc = jnp.dot(
        adj_ref[...], sup_ref[...], preferred_element_type=jnp.float32
    )
    xs = jnp.maximum(acc, 0.0) + bias_ref[...]
    gates = (
        jnp.dot(xs, wx_ref[...], preferred_element_type=jnp.float32)
        + jnp.dot(hx_ref[...], wh_ref[...], preferred_element_type=jnp.float32)
        + gb_ref[...]
    )
    ingate = jax.nn.sigmoid(gates[:, 0:h])
    forgetgate = jax.nn.sigmoid(gates[:, h:2 * h])
    cellgate = jnp.tanh(gates[:, 2 * h:3 * h])
    outgate = jax.nn.sigmoid(gates[:, 3 * h:4 * h])
    cy = cx_ref[...] * forgetgate + ingate * cellgate
    cy_ref[...] = cy
    hy_ref[...] = outgate * jnp.tanh(cy)


@jax.jit
def kernel(x, hx, cx, adj, gcn_weight, W_x2h, b_x2h, W_h2h, b_h2h, bias):
    n, d = x.shape
    h = hx.shape[1]

    wx_t = W_x2h.T
    wh_t = W_h2h.T
    gate_b = (b_x2h + b_h2h).reshape(1, 4 * h)
    bias2d = bias.reshape(1, h)

    bm = 400
    nm = n // bm

    hy, cy = pl.pallas_call(
        functools.partial(_main_kernel, h=h),
        grid=(nm,),
        in_specs=[
            pl.BlockSpec((bm, n), lambda i: (i, 0)),
            pl.BlockSpec((n, d), lambda i: (0, 0)),
            pl.BlockSpec((d, h), lambda i: (0, 0)),
            pl.BlockSpec((bm, h), lambda i: (i, 0)),
            pl.BlockSpec((bm, h), lambda i: (i, 0)),
            pl.BlockSpec((h, 4 * h), lambda i: (0, 0)),
            pl.BlockSpec((h, 4 * h), lambda i: (0, 0)),
            pl.BlockSpec((1, 4 * h), lambda i: (0, 0)),
            pl.BlockSpec((1, h), lambda i: (0, 0)),
        ],
        out_specs=[
            pl.BlockSpec((bm, h), lambda i: (i, 0)),
            pl.BlockSpec((bm, h), lambda i: (i, 0)),
        ],
        out_shape=[
            jax.ShapeDtypeStruct((n, h), jnp.float32),
            jax.ShapeDtypeStruct((n, h), jnp.float32),
        ],
        scratch_shapes=[pltpu.VMEM((n, h), jnp.float32)],
        compiler_params=pltpu.CompilerParams(
            dimension_semantics=("parallel",),
        ),
    )(adj, x, gcn_weight, hx, cx, wx_t, wh_t, gate_b, bias2d)

    return (hy, cy)
